# Initial kernel scaffold; baseline (speedup 1.0000x reference)
import jax
import jax.numpy as jnp
from jax.experimental import pallas as pl


def kernel(x, edge_index, batch, W1, b1, g1, be1, W2, b2, g2, be2, W3, b3, g3, be3, Wk1, bk1, Wk2, bk2, Wm1, bm1, Wm2, bm2):
    raise NotImplementedError("write your pallas kernel here")


# speedup vs baseline: 4.9225x; 4.9225x over previous
import jax
import jax.numpy as jnp
from jax import lax
from jax.experimental import pallas as pl
from jax.experimental.pallas import tpu as pltpu
from jax.experimental.pallas import tpu_sc as plsc

N = 50000
E = 800000
F = 22
H = 64
G = 256
EPS = 1e-5

NC = 2
NS = 16
CHUNK = 128
EP = 802816
NCH_F = EP // NS // CHUNK
NCH_D = EP // (NC * NS) // CHUNK
NROW = 50048
RPT = NROW // NS
W = 6
NG = 12
NPASS = NG // NC
HP = NG * W
BLK = 2000
NBLK = N // BLK


def _sc_degree_body(dst32, ones_h, zer_h, out, dst_v, ones_v, acc, sem):
    c = lax.axis_index("c")
    s = lax.axis_index("s")
    base = s * RPT
    pltpu.sync_copy(zer_h.at[pl.ds(base, RPT)], acc.at[pl.ds(base, RPT)])
    pltpu.sync_copy(ones_h, ones_v)
    pltpu.sync_copy(dst32.at[c * NS + s], dst_v)
    plsc.subcore_barrier()

    def step(j, _):
        pltpu.sync_copy(ones_v, acc.at[dst_v.at[j]], add=True)
        return None

    lax.fori_loop(0, NCH_D, step, None)
    plsc.subcore_barrier()
    pltpu.sync_copy(acc.at[pl.ds(base, RPT)],
                    out.at[c].at[pl.ds(base, RPT)])


def _sc_degree(dst32, ones_h, zer_h):
    return pl.kernel(
        _sc_degree_body,
        out_type=jax.ShapeDtypeStruct((NC, NROW, 2), jnp.float32),
        mesh=plsc.VectorSubcoreMesh(core_axis_name="c", subcore_axis_name="s"),
        compiler_params=pltpu.CompilerParams(use_tc_tiling_on_sc=False),
        scratch_types=[
            pltpu.VMEM((NCH_D, CHUNK), jnp.int32),
            pltpu.VMEM((CHUNK, 2), jnp.float32),
            pltpu.VMEM_SHARED((NROW, 2), jnp.float32),
            pltpu.SemaphoreType.DMA,
        ],
    )(dst32, ones_h, zer_h)


def _sc_scatter_body(tbl, src16, dst16, out, src_v, dst_v, acc, rows0, rows1,
                     sem0, sem1):
    c = lax.axis_index("c")
    s = lax.axis_index("s")
    base = s * RPT
    pltpu.sync_copy(src16.at[s], src_v)
    pltpu.sync_copy(dst16.at[s], dst_v)

    def run_pass(g):
        pltpu.sync_copy(tbl.at[g].at[pl.ds(base, RPT)],
                        acc.at[pl.ds(base, RPT)])
        plsc.subcore_barrier()
        pltpu.async_copy(tbl.at[g].at[src_v.at[0]], rows0, sem0)
        pltpu.async_copy(tbl.at[g].at[src_v.at[1]], rows1, sem1)

        def step(k, _):
            j0 = 2 * k
            j1 = j0 + 1
            pltpu.make_async_copy(tbl.at[g].at[src_v.at[j0]], rows0,
                                  sem0).wait()
            pltpu.sync_copy(rows0, acc.at[dst_v.at[j0]], add=True)

            @pl.when(j0 + 2 < NCH_F)
            def _():
                pltpu.async_copy(tbl.at[g].at[src_v.at[j0 + 2]], rows0, sem0)

            pltpu.make_async_copy(tbl.at[g].at[src_v.at[j1]], rows1,
                                  sem1).wait()
            pltpu.sync_copy(rows1, acc.at[dst_v.at[j1]], add=True)

            @pl.when(j1 + 2 < NCH_F)
            def _():
                pltpu.async_copy(tbl.at[g].at[src_v.at[j1 + 2]], rows1, sem1)

            return None

        lax.fori_loop(0, NCH_F // 2, step, None)
        plsc.subcore_barrier()
        pltpu.sync_copy(acc.at[pl.ds(base, RPT)],
                        out.at[g].at[pl.ds(base, RPT)])
        plsc.subcore_barrier()

    for p in range(NPASS):
        run_pass(NC * p + c)


def _sc_scatter(tbl, src16, dst16):
    return pl.kernel(
        _sc_scatter_body,
        out_type=jax.ShapeDtypeStruct((NG, NROW, W), jnp.float32),
        mesh=plsc.VectorSubcoreMesh(core_axis_name="c", subcore_axis_name="s"),
        compiler_params=pltpu.CompilerParams(use_tc_tiling_on_sc=False),
        scratch_types=[
            pltpu.VMEM((NCH_F, CHUNK), jnp.int32),
            pltpu.VMEM((NCH_F, CHUNK), jnp.int32),
            pltpu.VMEM_SHARED((NROW, W), jnp.float32),
            pltpu.VMEM((CHUNK, W), jnp.float32),
            pltpu.VMEM((CHUNK, W), jnp.float32),
            pltpu.SemaphoreType.DMA,
            pltpu.SemaphoreType.DMA,
        ],
    )(tbl, src16, dst16)


def _split_groups(y):
    return jnp.stack([y[:, g * W:(g + 1) * W] for g in range(NG)])


def _merge_groups(a):
    return jnp.concatenate([a[g] for g in range(NG)], axis=1)[:, :H]


def _prep1_body(x_ref, d_ref, w_ref, tbl_ref, dinv_ref):
    d = d_ref[...]
    deg = d[0, :, :1] + d[1, :, :1] + 1.0
    dinv = lax.rsqrt(deg)
    y = jnp.dot(x_ref[...], w_ref[...],
                preferred_element_type=jnp.float32) * dinv
    y = jnp.concatenate([y, jnp.zeros((BLK, HP - H), jnp.float32)], axis=1)
    tbl_ref[...] = _split_groups(y)
    dinv_ref[...] = dinv


def _prep1(x, degs, W1):
    return pl.pallas_call(
        _prep1_body,
        grid=(NBLK,),
        in_specs=[
            pl.BlockSpec((BLK, F), lambda i: (i, 0)),
            pl.BlockSpec((NC, BLK, 2), lambda i: (0, i, 0)),
            pl.BlockSpec((F, H), lambda i: (0, 0)),
        ],
        out_specs=[
            pl.BlockSpec((NG, BLK, W), lambda i: (0, i, 0)),
            pl.BlockSpec((BLK, 1), lambda i: (i, 0)),
        ],
        out_shape=[
            jax.ShapeDtypeStruct((NG, NROW, W), jnp.float32),
            jax.ShapeDtypeStruct((N, 1), jnp.float32),
        ],
    )(x, degs, W1)


def _stats_body(a_ref, dinv_ref, out_ref):
    z = _merge_groups(a_ref[...]) * dinv_ref[...]
    delta = jnp.stack([jnp.sum(z, 0), jnp.sum(z * z, 0)])

    @pl.when(pl.program_id(0) == 0)
    def _():
        out_ref[...] = jnp.zeros_like(out_ref)

    out_ref[...] += delta


def _stats(a, dinv):
    return pl.pallas_call(
        _stats_body,
        grid=(NBLK,),
        in_specs=[
            pl.BlockSpec((NG, BLK, W), lambda i: (0, i, 0)),
            pl.BlockSpec((BLK, 1), lambda i: (i, 0)),
        ],
        out_specs=pl.BlockSpec((2, H), lambda i: (0, 0)),
        out_shape=jax.ShapeDtypeStruct((2, H), jnp.float32),
    )(a, dinv)


def _bn_relu(a_ref, dinv_ref, st_ref, g_ref, be_ref):
    dinv = dinv_ref[...]
    z = _merge_groups(a_ref[...]) * dinv
    st = st_ref[...]
    m = st[0:1, :] * (1.0 / N)
    q = st[1:2, :] * (1.0 / N)
    var = q - m * m
    h = (z - m) * lax.rsqrt(var + EPS) * g_ref[...] + be_ref[...]
    return jnp.maximum(h, 0.0), dinv


def _apply_body(a_ref, dinv_ref, st_ref, g_ref, be_ref, w_ref, tbl_ref):
    h, dinv = _bn_relu(a_ref, dinv_ref, st_ref, g_ref, be_ref)
    y = jnp.dot(h, w_ref[...], preferred_element_type=jnp.float32) * dinv
    y = jnp.concatenate([y, jnp.zeros((BLK, HP - H), jnp.float32)], axis=1)
    tbl_ref[...] = _split_groups(y)


def _apply(a, dinv, st, g, be, Wn):
    return pl.pallas_call(
        _apply_body,
        grid=(NBLK,),
        in_specs=[
            pl.BlockSpec((NG, BLK, W), lambda i: (0, i, 0)),
            pl.BlockSpec((BLK, 1), lambda i: (i, 0)),
            pl.BlockSpec((2, H), lambda i: (0, 0)),
            pl.BlockSpec((1, H), lambda i: (0, 0)),
            pl.BlockSpec((1, H), lambda i: (0, 0)),
            pl.BlockSpec((H, H), lambda i: (0, 0)),
        ],
        out_specs=pl.BlockSpec((NG, BLK, W), lambda i: (0, i, 0)),
        out_shape=jax.ShapeDtypeStruct((NG, NROW, W), jnp.float32),
    )(a, dinv, st, g, be, Wn)


def _pool_body(a_ref, dinv_ref, st_ref, g_ref, be_ref, b_ref,
               sums_ref, cnts_ref):
    h, _ = _bn_relu(a_ref, dinv_ref, st_ref, g_ref, be_ref)
    seg = b_ref[...]
    oh = (seg == lax.broadcasted_iota(jnp.int32, (1, G), 1)).astype(
        jnp.float32)
    sums_d = lax.dot_general(oh, h, (((0,), (0,)), ((), ())),
                             preferred_element_type=jnp.float32)
    cnts_d = jnp.sum(oh, axis=0)[:, None]

    @pl.when(pl.program_id(0) == 0)
    def _():
        sums_ref[...] = jnp.zeros_like(sums_ref)
        cnts_ref[...] = jnp.zeros_like(cnts_ref)

    sums_ref[...] += sums_d
    cnts_ref[...] += cnts_d


def _pool(a, dinv, st, g, be, batch2):
    return pl.pallas_call(
        _pool_body,
        grid=(NBLK,),
        in_specs=[
            pl.BlockSpec((NG, BLK, W), lambda i: (0, i, 0)),
            pl.BlockSpec((BLK, 1), lambda i: (i, 0)),
            pl.BlockSpec((2, H), lambda i: (0, 0)),
            pl.BlockSpec((1, H), lambda i: (0, 0)),
            pl.BlockSpec((1, H), lambda i: (0, 0)),
            pl.BlockSpec((BLK, 1), lambda i: (i, 0)),
        ],
        out_specs=[
            pl.BlockSpec((G, H), lambda i: (0, 0)),
            pl.BlockSpec((G, 1), lambda i: (0, 0)),
        ],
        out_shape=[
            jax.ShapeDtypeStruct((G, H), jnp.float32),
            jax.ShapeDtypeStruct((G, 1), jnp.float32),
        ],
    )(a, dinv, st, g, be, batch2)


def _heads_body(sums_ref, cnts_ref, wk1_ref, bk1_ref, wk2_ref, bk2_ref,
                wm1_ref, bm1_ref, wm2_ref, bm2_ref, kcat_ref, km_ref):
    pooled = sums_ref[...] / jnp.maximum(cnts_ref[...], 1.0)

    def head(w1, b1, w2, b2):
        t = jnp.maximum(
            jnp.dot(pooled, w1[...], preferred_element_type=jnp.float32)
            + b1[...], 0.0)
        return jnp.dot(t, w2[...],
                       preferred_element_type=jnp.float32) + b2[...]

    kcat_ref[...] = head(wk1_ref, bk1_ref, wk2_ref, bk2_ref)
    km_ref[...] = head(wm1_ref, bm1_ref, wm2_ref, bm2_ref)


def _heads(sums, cnts, Wk1, bk1, Wk2, bk2, Wm1, bm1, Wm2, bm2):
    return pl.pallas_call(
        _heads_body,
        out_shape=[
            jax.ShapeDtypeStruct((G, 1), jnp.float32),
            jax.ShapeDtypeStruct((G, 1), jnp.float32),
        ],
    )(sums, cnts, Wk1, bk1, Wk2, bk2, Wm1, bm1, Wm2, bm2)


def kernel(x, edge_index, batch, W1, b1, g1, be1, W2, b2, g2, be2,
           W3, b3, g3, be3, Wk1, bk1, Wk2, bk2, Wm1, bm1, Wm2, bm2):
    src = edge_index[0].astype(jnp.int32)
    dst = edge_index[1].astype(jnp.int32)
    pad = EP - E
    srcp = jnp.concatenate([src, jnp.zeros((pad,), jnp.int32)])
    dstp = jnp.concatenate([dst, jnp.full((pad,), N, jnp.int32)])
    src16 = srcp.reshape(NS, NCH_F, CHUNK)
    dst16 = dstp.reshape(NS, NCH_F, CHUNK)
    dst32 = dstp.reshape(NC * NS, NCH_D, CHUNK)
    ones_h = jnp.ones((CHUNK, 2), jnp.float32)
    zer_h = jnp.zeros((NROW, 2), jnp.float32)
    batch2 = batch.astype(jnp.int32).reshape(N, 1)
    g1r, be1r = g1.reshape(1, H), be1.reshape(1, H)
    g2r, be2r = g2.reshape(1, H), be2.reshape(1, H)
    g3r, be3r = g3.reshape(1, H), be3.reshape(1, H)

    degs = _sc_degree(dst32, ones_h, zer_h)
    tbl, dinv = _prep1(x, degs, W1)

    a = _sc_scatter(tbl, src16, dst16)
    st1 = _stats(a, dinv)
    tbl = _apply(a, dinv, st1, g1r, be1r, W2)

    a = _sc_scatter(tbl, src16, dst16)
    st2 = _stats(a, dinv)
    tbl = _apply(a, dinv, st2, g2r, be2r, W3)

    a = _sc_scatter(tbl, src16, dst16)
    st3 = _stats(a, dinv)
    sums, cnts = _pool(a, dinv, st3, g3r, be3r, batch2)

    kcat, km = _heads(sums, cnts, Wk1, bk1.reshape(1, H), Wk2,
                      bk2.reshape(1, 1), Wm1, bm1.reshape(1, H), Wm2,
                      bm2.reshape(1, 1))
    return (kcat, km)
